# baseline (speedup 1.0000x reference)
import jax
import jax.numpy as jnp
from jax.experimental import pallas as pl
from jax.experimental.pallas import tpu as pltpu

T = 32768
H = 768
E = 64
TOP_K = 2
CH = 2048
NCH = T // CH
NBUF = 3


def _compute_chunk(x, w, b):
    logits = jax.lax.dot_general(
        x, w, (((1,), (0,)), ((), ())),
        preferred_element_type=jnp.float32) + b
    m = jnp.max(logits, axis=1, keepdims=True)
    ex = jnp.exp(logits - m)
    aff = ex / jnp.sum(ex, axis=1, keepdims=True)
    iota = jax.lax.broadcasted_iota(jnp.int32, (CH, E), 1)
    big = jnp.int32(E)
    top1 = jnp.max(aff, axis=1, keepdims=True)
    idx1 = jnp.min(jnp.where(aff == top1, iota, big), axis=1, keepdims=True)
    masked = jnp.where(iota == idx1, -jnp.inf, aff)
    top2 = jnp.max(masked, axis=1, keepdims=True)
    idx2 = jnp.min(jnp.where(masked == top2, iota, big), axis=1, keepdims=True)
    idx = jnp.concatenate([idx1, idx2], axis=1)
    return logits, aff, idx


def _router(x_hbm, w_hbm, b_hbm, logits_hbm, aff_hbm, idx_hbm,
            xb, wb, bb, lg, af, ix, in_sems, out_sems, w_sem):
    cw = pltpu.make_async_copy(w_hbm, wb, w_sem)
    cw.start()
    cb = pltpu.make_async_copy(b_hbm, bb, w_sem)
    cb.start()
    for j in range(min(NBUF, NCH)):
        pltpu.make_async_copy(
            x_hbm.at[pl.ds(j * CH, CH)], xb.at[j], in_sems.at[j]).start()
    cw.wait()
    cb.wait()

    for i in range(NCH):
        sl = i % NBUF
        pltpu.make_async_copy(
            x_hbm.at[pl.ds(i * CH, CH)], xb.at[sl], in_sems.at[sl]).wait()

        if i >= NBUF:
            prev = (i - NBUF) * CH
            pltpu.make_async_copy(
                lg.at[sl], logits_hbm.at[pl.ds(prev, CH)], out_sems.at[sl]).wait()
            pltpu.make_async_copy(
                af.at[sl], aff_hbm.at[pl.ds(prev, CH)], out_sems.at[sl]).wait()
            pltpu.make_async_copy(
                ix.at[sl], idx_hbm.at[pl.ds(prev, CH)], out_sems.at[sl]).wait()

        logits, aff, idx = _compute_chunk(xb[sl], wb[...], bb[...])
        lg[sl] = logits
        af[sl] = aff
        ix[sl] = idx

        nxt = i + NBUF
        if nxt < NCH:
            pltpu.make_async_copy(
                x_hbm.at[pl.ds(nxt * CH, CH)], xb.at[sl],
                in_sems.at[sl]).start()

        base = i * CH
        pltpu.make_async_copy(
            lg.at[sl], logits_hbm.at[pl.ds(base, CH)], out_sems.at[sl]).start()
        pltpu.make_async_copy(
            af.at[sl], aff_hbm.at[pl.ds(base, CH)], out_sems.at[sl]).start()
        pltpu.make_async_copy(
            ix.at[sl], idx_hbm.at[pl.ds(base, CH)], out_sems.at[sl]).start()

    for i in range(max(NCH - NBUF, 0), NCH):
        sl = i % NBUF
        base = i * CH
        pltpu.make_async_copy(
            lg.at[sl], logits_hbm.at[pl.ds(base, CH)], out_sems.at[sl]).wait()
        pltpu.make_async_copy(
            af.at[sl], aff_hbm.at[pl.ds(base, CH)], out_sems.at[sl]).wait()
        pltpu.make_async_copy(
            ix.at[sl], idx_hbm.at[pl.ds(base, CH)], out_sems.at[sl]).wait()


def kernel(hidden_states, W, b):
    b2 = b.reshape(1, E)
    logits, aff, idx = pl.pallas_call(
        _router,
        in_specs=[
            pl.BlockSpec(memory_space=pl.ANY),
            pl.BlockSpec(memory_space=pl.ANY),
            pl.BlockSpec(memory_space=pl.ANY),
        ],
        out_specs=[
            pl.BlockSpec(memory_space=pl.ANY),
            pl.BlockSpec(memory_space=pl.ANY),
            pl.BlockSpec(memory_space=pl.ANY),
        ],
        out_shape=[
            jax.ShapeDtypeStruct((T, E), jnp.float32),
            jax.ShapeDtypeStruct((T, E), jnp.float32),
            jax.ShapeDtypeStruct((T, TOP_K), jnp.int32),
        ],
        scratch_shapes=[
            pltpu.VMEM((NBUF, CH, H), jnp.float32),
            pltpu.VMEM((H, E), jnp.float32),
            pltpu.VMEM((1, E), jnp.float32),
            pltpu.VMEM((NBUF, CH, E), jnp.float32),
            pltpu.VMEM((NBUF, CH, E), jnp.float32),
            pltpu.VMEM((NBUF, CH, TOP_K), jnp.int32),
            pltpu.SemaphoreType.DMA((NBUF,)),
            pltpu.SemaphoreType.DMA((NBUF,)),
            pltpu.SemaphoreType.DMA,
        ],
        compiler_params=pltpu.CompilerParams(
            vmem_limit_bytes=100 * 1024 * 1024,
        ),
    )(hidden_states, W, b2)
    return (logits, aff, idx)
